# baseline (speedup 1.0000x reference)
import jax
import jax.numpy as jnp
from jax.experimental import pallas as pl
from jax.experimental.pallas import tpu as pltpu

N = 4096
H = 24
Z = 64
BM = 512
NB = N // BM
BR = 512
NR = N // BR


def _gcn_body(x0_ref, adj_ref, w1_ref, b1_ref, w2_ref, b2_ref,
              out_ref, y1_ref, y2_ref, adjb_ref):
    i = pl.program_id(0)

    @pl.when(i == 0)
    def _init():
        y1 = jax.lax.dot_general(
            x0_ref[...], w1_ref[...],
            dimension_numbers=(((0,), (0,)), ((), ())),
            preferred_element_type=jnp.float32)
        y1_ref[...] = y1.astype(jnp.bfloat16)

    off = pl.multiple_of(i * BM, BM)
    ab = adj_ref[...].astype(jnp.bfloat16)
    adjb_ref[pl.ds(off, BM), :] = ab
    h1 = jnp.dot(ab, y1_ref[...], preferred_element_type=jnp.float32)
    s1 = jnp.maximum(h1 + b1_ref[...][None, :], 0.0)
    y2_ref[pl.ds(off, BM), :] = jnp.dot(
        s1.astype(jnp.bfloat16), w2_ref[...].astype(jnp.bfloat16),
        preferred_element_type=jnp.float32,
    ).astype(jnp.bfloat16)

    @pl.when(i == NB - 1)
    def _final():
        b2v = b2_ref[...][:, None]
        outT = jax.lax.dot_general(
            y2_ref[...], adjb_ref[...],
            dimension_numbers=(((0,), (1,)), ((), ())),
            preferred_element_type=jnp.float32)
        out_ref[...] = outT + b2v


def _gcn(x0, Adj, W1, b1, W2, b2, interpret=False):
    return pl.pallas_call(
        _gcn_body,
        grid=(NB,),
        in_specs=[
            pl.BlockSpec((H, N), lambda i: (0, 0)),
            pl.BlockSpec((BM, N), lambda i: (i, 0)),
            pl.BlockSpec((H, Z), lambda i: (0, 0)),
            pl.BlockSpec((Z,), lambda i: (0,)),
            pl.BlockSpec((Z, H), lambda i: (0, 0)),
            pl.BlockSpec((H,), lambda i: (0,)),
        ],
        out_specs=pl.BlockSpec((H, N), lambda i: (0, 0)),
        out_shape=jax.ShapeDtypeStruct((H, N), jnp.float32),
        scratch_shapes=[
            pltpu.VMEM((N, Z), jnp.bfloat16),
            pltpu.VMEM((N, H), jnp.bfloat16),
            pltpu.VMEM((N, N), jnp.bfloat16),
        ],
        interpret=interpret,
    )(x0, Adj, W1, b1, W2, b2)


def kernel(X, A_q, A_h, Adj, W1, b1, W2, b2):
    out = _gcn(X[0], Adj, W1, b1, W2, b2)
    return out[None]
